# baseline (speedup 1.0000x reference)
import functools

import jax
import jax.numpy as jnp
from jax import lax
from jax.experimental import pallas as pl
from jax.experimental.pallas import tpu as pltpu
from jax.experimental.pallas import tpu_sc as plsc

H = 1024
DFF = 512
E = 8
K = 2
SCALE = 2.5
T = 2048
B = 256
NB = 32
P = NB * B


def _router_body(x_ref, gw_ref, w_ref, i_ref):
    x = x_ref[...]
    logits = lax.dot_general(x, gw_ref[...], (((1,), (1,)), ((), ())),
                             preferred_element_type=jnp.float32)
    v = jax.nn.sigmoid(logits)
    lane = lax.broadcasted_iota(jnp.int32, v.shape, 1)
    m1 = jnp.max(v, axis=1, keepdims=True)
    i1 = jnp.min(jnp.where(v == m1, lane, E), axis=1, keepdims=True)
    vm = jnp.where(lane == i1, -jnp.inf, v)
    m2 = jnp.max(vm, axis=1, keepdims=True)
    i2 = jnp.min(jnp.where(vm == m2, lane, E), axis=1, keepdims=True)
    s = m1 + m2 + 1e-6
    w_ref[...] = jnp.concatenate([m1 / s, m2 / s], axis=1) * SCALE
    i_ref[...] = jnp.concatenate([i1, i2], axis=1)


def _router(x, gate_w):
    return pl.pallas_call(
        _router_body,
        out_shape=(jax.ShapeDtypeStruct((T, K), jnp.float32),
                   jax.ShapeDtypeStruct((T, K), jnp.int32)),
    )(x, gate_w)


def _make_sc_gather(n_rows, n_chunks):
    info = plsc.get_sparse_core_info()
    nw = info.num_cores * info.num_subcores
    n_w = n_rows // nw
    chunk = n_w // n_chunks
    mesh = plsc.VectorSubcoreMesh(core_axis_name="c", subcore_axis_name="s")

    @functools.partial(
        pl.kernel, mesh=mesh,
        out_type=jax.ShapeDtypeStruct((n_rows, H), jnp.float32),
        scratch_types=[
            pltpu.VMEM((n_w,), jnp.int32),
            pltpu.VMEM((chunk, H), jnp.float32),
            pltpu.SemaphoreType.DMA,
        ],
    )
    def gather_kernel(src_hbm, idx_hbm, out_hbm, idx_v, rows_v, sem):
        wid = lax.axis_index("s") * info.num_cores + lax.axis_index("c")
        base = wid * n_w
        pltpu.sync_copy(idx_hbm.at[pl.ds(base, n_w)], idx_v)
        for c in range(n_chunks):
            pltpu.async_copy(
                src_hbm.at[idx_v.at[pl.ds(c * chunk, chunk)]], rows_v, sem
            ).wait()
            pltpu.sync_copy(rows_v, out_hbm.at[pl.ds(base + c * chunk, chunk)])

    return gather_kernel


def _gmm_body(be_ref, x_ref, gw_ref, uw_ref, dw_ref, y_ref):
    del be_ref
    xb = x_ref[...].astype(jnp.bfloat16)
    g = lax.dot_general(xb, gw_ref[0], (((1,), (1,)), ((), ())),
                        preferred_element_type=jnp.float32)
    u = lax.dot_general(xb, uw_ref[0], (((1,), (1,)), ((), ())),
                        preferred_element_type=jnp.float32)
    h = (jax.nn.silu(g) * u).astype(jnp.bfloat16)
    y_ref[...] = lax.dot_general(h, dw_ref[0], (((1,), (1,)), ((), ())),
                                 preferred_element_type=jnp.float32)


def _gmm(block_expert, xg, gw, uw, dw):
    grid_spec = pltpu.PrefetchScalarGridSpec(
        num_scalar_prefetch=1,
        grid=(NB,),
        in_specs=[
            pl.BlockSpec((B, H), lambda i, be: (i, 0)),
            pl.BlockSpec((1, DFF, H), lambda i, be: (be[i], 0, 0)),
            pl.BlockSpec((1, DFF, H), lambda i, be: (be[i], 0, 0)),
            pl.BlockSpec((1, H, DFF), lambda i, be: (be[i], 0, 0)),
        ],
        out_specs=pl.BlockSpec((B, H), lambda i, be: (i, 0)),
    )
    return pl.pallas_call(
        _gmm_body,
        grid_spec=grid_spec,
        out_shape=jax.ShapeDtypeStruct((P, H), jnp.float32),
    )(block_expert, xg, gw, uw, dw)


def _combine_body(g_ref, w_ref, o_ref):
    w = w_ref[...]
    o_ref[...] = (w[:, 0:1] * g_ref[:, :H]
                  + w[:, 1:2] * g_ref[:, H:2 * H]
                  + g_ref[:, 2 * H:])


def _combine(g, topk_w):
    bt = 256
    return pl.pallas_call(
        _combine_body,
        grid=(T // bt,),
        in_specs=[pl.BlockSpec((bt, 3 * H), lambda i: (i, 0)),
                  pl.BlockSpec((bt, K), lambda i: (i, 0))],
        out_specs=pl.BlockSpec((bt, H), lambda i: (i, 0)),
        out_shape=jax.ShapeDtypeStruct((T, H), jnp.float32),
    )(g, topk_w)


def kernel(hidden_states, gate_w, shared_gate_w, shared_up_w, shared_down_w,
           expert_gate_w, expert_up_w, expert_down_w):
    orig_shape = hidden_states.shape
    x = hidden_states.reshape(-1, H)

    topk_w, topk_i = _router(x, gate_w)

    flat_e = topk_i.reshape(-1)
    onehot = (flat_e[:, None] == jnp.arange(E)[None, :]).astype(jnp.int32)
    incl = jnp.cumsum(onehot, axis=0)
    counts = incl[-1]
    pos = incl[jnp.arange(T * K), flat_e] - 1
    pad_counts = ((counts + B - 1) // B) * B
    pad_off = jnp.concatenate([jnp.zeros((1,), jnp.int32),
                               jnp.cumsum(pad_counts)]).astype(jnp.int32)
    routed_end = pad_off[E]
    dest = pad_off[flat_e] + pos

    tok = jnp.arange(T, dtype=jnp.int32)
    sorted_token = jnp.zeros((P,), jnp.int32).at[dest].set(
        jnp.arange(T * K, dtype=jnp.int32) // K)
    sorted_token = lax.dynamic_update_slice(sorted_token, tok, (routed_end,))

    b_start = jnp.arange(NB, dtype=jnp.int32) * B
    block_expert = jnp.sum(
        (b_start[:, None] >= pad_off[None, 1:E + 1]).astype(jnp.int32), axis=1)

    d_full = jnp.concatenate(
        [dest.reshape(T, K), (routed_end + tok)[:, None]], axis=1).reshape(-1)

    xg = _make_sc_gather(P, 4)(x, sorted_token)

    gw = jnp.concatenate([expert_gate_w, shared_gate_w[None]], axis=0)
    uw = jnp.concatenate([expert_up_w, shared_up_w[None]], axis=0)
    dw = jnp.concatenate([expert_down_w, shared_down_w[None]], axis=0)
    yg = _gmm(block_expert, xg,
              gw.astype(jnp.bfloat16), uw.astype(jnp.bfloat16),
              dw.astype(jnp.bfloat16))

    g = _make_sc_gather(T * 3, 2)(yg, d_full.astype(jnp.int32))

    out = _combine(g.reshape(T, 3 * H), topk_w)
    return out.reshape(orig_shape)


# speedup vs baseline: 1.3941x; 1.3941x over previous
import functools

import jax
import jax.numpy as jnp
from jax import lax
from jax.experimental import pallas as pl
from jax.experimental.pallas import tpu as pltpu
from jax.experimental.pallas import tpu_sc as plsc

H = 1024
DFF = 512
E = 8
K = 2
SCALE = 2.5
T = 2048
B = 256
NB = 32
P = NB * B


def _router_body(x_ref, gw_ref, w_ref, i_ref):
    x = x_ref[...]
    logits = lax.dot_general(x, gw_ref[...], (((1,), (1,)), ((), ())),
                             preferred_element_type=jnp.float32)
    v = jax.nn.sigmoid(logits)
    lane = lax.broadcasted_iota(jnp.int32, v.shape, 1)
    m1 = jnp.max(v, axis=1, keepdims=True)
    i1 = jnp.min(jnp.where(v == m1, lane, E), axis=1, keepdims=True)
    vm = jnp.where(lane == i1, -jnp.inf, v)
    m2 = jnp.max(vm, axis=1, keepdims=True)
    i2 = jnp.min(jnp.where(vm == m2, lane, E), axis=1, keepdims=True)
    s = m1 + m2 + 1e-6
    w_ref[...] = jnp.concatenate([m1 / s, m2 / s], axis=1) * SCALE
    i_ref[...] = jnp.concatenate([i1, i2], axis=1)


def _router(x, gate_w):
    return pl.pallas_call(
        _router_body,
        out_shape=(jax.ShapeDtypeStruct((T, K), jnp.float32),
                   jax.ShapeDtypeStruct((T, K), jnp.int32)),
    )(x, gate_w)


def _make_sc_gather(n_rows, n_chunks, name):
    info = plsc.get_sparse_core_info()
    nw = info.num_cores * info.num_subcores
    n_w = n_rows // nw
    chunk = n_w // n_chunks
    mesh = plsc.VectorSubcoreMesh(core_axis_name="c", subcore_axis_name="s")

    @functools.partial(
        pl.kernel, mesh=mesh, name=name,
        out_type=jax.ShapeDtypeStruct((n_rows, H), jnp.float32),
        scratch_types=[
            pltpu.VMEM((n_w,), jnp.int32),
            pltpu.VMEM((chunk, H), jnp.float32),
            pltpu.SemaphoreType.DMA,
        ],
    )
    def gather_kernel(src_hbm, idx_hbm, out_hbm, idx_v, rows_v, sem):
        wid = lax.axis_index("s") * info.num_cores + lax.axis_index("c")
        base = wid * n_w
        pltpu.sync_copy(idx_hbm.at[pl.ds(base, n_w)], idx_v)
        for c in range(n_chunks):
            pltpu.async_copy(
                src_hbm.at[idx_v.at[pl.ds(c * chunk, chunk)]], rows_v, sem
            ).wait()
            pltpu.sync_copy(rows_v, out_hbm.at[pl.ds(base + c * chunk, chunk)])

    return gather_kernel


def _gmm_body(be_ref, x_ref, gw_ref, uw_ref, dw_ref, y_ref):
    del be_ref
    xb = x_ref[...].astype(jnp.bfloat16)
    g = lax.dot_general(xb, gw_ref[0], (((1,), (1,)), ((), ())),
                        preferred_element_type=jnp.float32)
    u = lax.dot_general(xb, uw_ref[0], (((1,), (1,)), ((), ())),
                        preferred_element_type=jnp.float32)
    h = (jax.nn.silu(g) * u).astype(jnp.bfloat16)
    y_ref[...] = lax.dot_general(h, dw_ref[0], (((1,), (1,)), ((), ())),
                                 preferred_element_type=jnp.float32)


def _gmm(block_expert, xg, gw, uw, dw):
    grid_spec = pltpu.PrefetchScalarGridSpec(
        num_scalar_prefetch=1,
        grid=(NB,),
        in_specs=[
            pl.BlockSpec((B, H), lambda i, be: (i, 0)),
            pl.BlockSpec((1, DFF, H), lambda i, be: (be[i], 0, 0)),
            pl.BlockSpec((1, DFF, H), lambda i, be: (be[i], 0, 0)),
            pl.BlockSpec((1, H, DFF), lambda i, be: (be[i], 0, 0)),
        ],
        out_specs=pl.BlockSpec((B, H), lambda i, be: (i, 0)),
    )
    return pl.pallas_call(
        _gmm_body,
        grid_spec=grid_spec,
        out_shape=jax.ShapeDtypeStruct((P, H), jnp.float32),
    )(block_expert, xg, gw, uw, dw)


def _combine_body(g_ref, w_ref, o_ref):
    w = w_ref[...]
    o_ref[...] = (w[:, 0:1] * g_ref[:, :H]
                  + w[:, 1:2] * g_ref[:, H:2 * H]
                  + g_ref[:, 2 * H:])


def _combine(g, topk_w):
    bt = 256
    return pl.pallas_call(
        _combine_body,
        grid=(T // bt,),
        in_specs=[pl.BlockSpec((bt, 3 * H), lambda i: (i, 0)),
                  pl.BlockSpec((bt, K), lambda i: (i, 0))],
        out_specs=pl.BlockSpec((bt, H), lambda i: (i, 0)),
        out_shape=jax.ShapeDtypeStruct((T, H), jnp.float32),
    )(g, topk_w)


def kernel(hidden_states, gate_w, shared_gate_w, shared_up_w, shared_down_w,
           expert_gate_w, expert_up_w, expert_down_w):
    orig_shape = hidden_states.shape
    x = hidden_states.reshape(-1, H)

    topk_w, topk_i = _router(x, gate_w)

    flat_e = topk_i.reshape(-1)
    onehot = (flat_e[:, None] == jnp.arange(E)[None, :]).astype(jnp.int32)
    incl = jnp.cumsum(onehot, axis=0)
    counts = incl[-1]
    pos = incl[jnp.arange(T * K), flat_e] - 1
    pad_counts = ((counts + B - 1) // B) * B
    pad_off = jnp.concatenate([jnp.zeros((1,), jnp.int32),
                               jnp.cumsum(pad_counts)]).astype(jnp.int32)
    routed_end = pad_off[E]
    dest = pad_off[flat_e] + pos

    tok = jnp.arange(T, dtype=jnp.int32)
    sorted_token = (jnp.arange(P, dtype=jnp.int32) % T).at[dest].set(
        jnp.arange(T * K, dtype=jnp.int32) // K)
    sorted_token = lax.dynamic_update_slice(sorted_token, tok, (routed_end,))

    b_start = jnp.arange(NB, dtype=jnp.int32) * B
    block_expert = jnp.sum(
        (b_start[:, None] >= pad_off[None, 1:E + 1]).astype(jnp.int32), axis=1)

    d_full = jnp.concatenate(
        [dest.reshape(T, K), (routed_end + tok)[:, None]], axis=1).reshape(-1)

    xg = _make_sc_gather(P, 4, "sc_gather_dispatch")(x, sorted_token)

    gw = jnp.concatenate([expert_gate_w, shared_gate_w[None]], axis=0)
    uw = jnp.concatenate([expert_up_w, shared_up_w[None]], axis=0)
    dw = jnp.concatenate([expert_down_w, shared_down_w[None]], axis=0)
    yg = _gmm(block_expert, xg,
              gw.astype(jnp.bfloat16), uw.astype(jnp.bfloat16),
              dw.astype(jnp.bfloat16))

    g = _make_sc_gather(T * 3, 2, "sc_gather_combine")(yg, d_full.astype(jnp.int32))

    out = _combine(g.reshape(T, 3 * H), topk_w)
    return out.reshape(orig_shape)


# speedup vs baseline: 1.7436x; 1.2507x over previous
import functools

import jax
import jax.numpy as jnp
from jax import lax
from jax.experimental import pallas as pl
from jax.experimental.pallas import tpu as pltpu
from jax.experimental.pallas import tpu_sc as plsc

H = 1024
DFF = 512
E = 8
K = 2
SCALE = 2.5
T = 2048
B = 256
NBR = 24
PR = NBR * B


def _router_body(x_ref, gw_ref, w_ref, i_ref):
    x = x_ref[...]
    logits = lax.dot_general(x, gw_ref[...], (((1,), (1,)), ((), ())),
                             preferred_element_type=jnp.float32)
    v = jax.nn.sigmoid(logits)
    lane = lax.broadcasted_iota(jnp.int32, v.shape, 1)
    m1 = jnp.max(v, axis=1, keepdims=True)
    i1 = jnp.min(jnp.where(v == m1, lane, E), axis=1, keepdims=True)
    vm = jnp.where(lane == i1, -jnp.inf, v)
    m2 = jnp.max(vm, axis=1, keepdims=True)
    i2 = jnp.min(jnp.where(vm == m2, lane, E), axis=1, keepdims=True)
    s = m1 + m2 + 1e-6
    w_ref[...] = jnp.concatenate([m1 / s, m2 / s], axis=1) * SCALE
    i_ref[...] = jnp.concatenate([i1, i2], axis=1)


def _router(x, gate_w):
    return pl.pallas_call(
        _router_body,
        out_shape=(jax.ShapeDtypeStruct((T, K), jnp.float32),
                   jax.ShapeDtypeStruct((T, K), jnp.int32)),
    )(x, gate_w)


def _make_sc_gather(n_rows, n_chunks, name):
    info = plsc.get_sparse_core_info()
    nw = info.num_cores * info.num_subcores
    n_w = n_rows // nw
    chunk = n_w // n_chunks
    mesh = plsc.VectorSubcoreMesh(core_axis_name="c", subcore_axis_name="s")

    @functools.partial(
        pl.kernel, mesh=mesh, name=name,
        out_type=jax.ShapeDtypeStruct((n_rows, H), jnp.float32),
        scratch_types=[
            pltpu.VMEM((n_w,), jnp.int32),
            pltpu.VMEM((chunk, H), jnp.float32),
            pltpu.SemaphoreType.DMA,
        ],
    )
    def gather_kernel(src_hbm, idx_hbm, out_hbm, idx_v, rows_v, sem):
        wid = lax.axis_index("s") * info.num_cores + lax.axis_index("c")
        base = wid * n_w
        pltpu.sync_copy(idx_hbm.at[pl.ds(base, n_w)], idx_v)
        for c in range(n_chunks):
            pltpu.async_copy(
                src_hbm.at[idx_v.at[pl.ds(c * chunk, chunk)]], rows_v, sem
            ).wait()
            pltpu.sync_copy(rows_v, out_hbm.at[pl.ds(base + c * chunk, chunk)])

    return gather_kernel


def _mlp_body(x_ref, gw_ref, uw_ref, dw_ref, y_ref):
    xb = x_ref[...].astype(jnp.bfloat16)
    g = lax.dot_general(xb, gw_ref[0], (((1,), (1,)), ((), ())),
                        preferred_element_type=jnp.float32)
    u = lax.dot_general(xb, uw_ref[0], (((1,), (1,)), ((), ())),
                        preferred_element_type=jnp.float32)
    h = (jax.nn.silu(g) * u).astype(jnp.bfloat16)
    y_ref[...] = lax.dot_general(h, dw_ref[0], (((1,), (1,)), ((), ())),
                                 preferred_element_type=jnp.float32)


def _gmm(block_expert, xg, gw, uw, dw):
    grid_spec = pltpu.PrefetchScalarGridSpec(
        num_scalar_prefetch=1,
        grid=(NBR,),
        in_specs=[
            pl.BlockSpec((B, H), lambda i, be: (i, 0)),
            pl.BlockSpec((1, DFF, H), lambda i, be: (be[i], 0, 0)),
            pl.BlockSpec((1, DFF, H), lambda i, be: (be[i], 0, 0)),
            pl.BlockSpec((1, H, DFF), lambda i, be: (be[i], 0, 0)),
        ],
        out_specs=pl.BlockSpec((B, H), lambda i, be: (i, 0)),
    )
    body = lambda be_ref, x, g, u, d, y: _mlp_body(x, g, u, d, y)
    return pl.pallas_call(
        body,
        grid_spec=grid_spec,
        out_shape=jax.ShapeDtypeStruct((PR, H), jnp.float32),
    )(block_expert, xg, gw, uw, dw)


def _shared_mlp(x, sgw, suw, sdw):
    return pl.pallas_call(
        _mlp_body,
        grid=(T // B,),
        in_specs=[
            pl.BlockSpec((B, H), lambda i: (i, 0)),
            pl.BlockSpec((1, DFF, H), lambda i: (0, 0, 0)),
            pl.BlockSpec((1, DFF, H), lambda i: (0, 0, 0)),
            pl.BlockSpec((1, H, DFF), lambda i: (0, 0, 0)),
        ],
        out_specs=pl.BlockSpec((B, H), lambda i: (i, 0)),
        out_shape=jax.ShapeDtypeStruct((T, H), jnp.float32),
    )(x, sgw, suw, sdw)


def _combine_body(g_ref, sh_ref, w_ref, o_ref):
    w = w_ref[...]
    o_ref[...] = (w[:, 0:1] * g_ref[:, :H]
                  + w[:, 1:2] * g_ref[:, H:]
                  + sh_ref[...])


def _combine(g, sh, topk_w):
    bt = 256
    return pl.pallas_call(
        _combine_body,
        grid=(T // bt,),
        in_specs=[pl.BlockSpec((bt, K * H), lambda i: (i, 0)),
                  pl.BlockSpec((bt, H), lambda i: (i, 0)),
                  pl.BlockSpec((bt, K), lambda i: (i, 0))],
        out_specs=pl.BlockSpec((bt, H), lambda i: (i, 0)),
        out_shape=jax.ShapeDtypeStruct((T, H), jnp.float32),
    )(g, sh, topk_w)


def kernel(hidden_states, gate_w, shared_gate_w, shared_up_w, shared_down_w,
           expert_gate_w, expert_up_w, expert_down_w):
    orig_shape = hidden_states.shape
    x = hidden_states.reshape(-1, H)

    topk_w, topk_i = _router(x, gate_w)

    flat_e = topk_i.reshape(-1)
    onehot = (flat_e[:, None] == jnp.arange(E)[None, :]).astype(jnp.int32)
    incl = jnp.cumsum(onehot, axis=0)
    counts = incl[-1]
    pos = incl[jnp.arange(T * K), flat_e] - 1
    pad_counts = ((counts + B - 1) // B) * B
    pad_off = jnp.concatenate([jnp.zeros((1,), jnp.int32),
                               jnp.cumsum(pad_counts)]).astype(jnp.int32)
    dest = pad_off[flat_e] + pos

    sorted_token = (jnp.arange(PR, dtype=jnp.int32) % T).at[dest].set(
        jnp.arange(T * K, dtype=jnp.int32) // K)

    b_start = jnp.arange(NBR, dtype=jnp.int32) * B
    block_expert = jnp.minimum(
        jnp.sum((b_start[:, None] >= pad_off[None, 1:E + 1]).astype(jnp.int32),
                axis=1), E - 1)

    xg = _make_sc_gather(PR, 2, "sc_gather_dispatch")(x, sorted_token)

    yg = _gmm(block_expert, xg,
              expert_gate_w.astype(jnp.bfloat16),
              expert_up_w.astype(jnp.bfloat16),
              expert_down_w.astype(jnp.bfloat16))

    sh = _shared_mlp(x, shared_gate_w[None].astype(jnp.bfloat16),
                     shared_up_w[None].astype(jnp.bfloat16),
                     shared_down_w[None].astype(jnp.bfloat16))

    g = _make_sc_gather(T * K, 2, "sc_gather_combine")(
        yg, dest.reshape(T, K).reshape(-1))

    out = _combine(g.reshape(T, K * H), sh, topk_w)
    return out.reshape(orig_shape)


# speedup vs baseline: 1.9004x; 1.0899x over previous
import functools

import jax
import jax.numpy as jnp
from jax import lax
from jax.experimental import pallas as pl
from jax.experimental.pallas import tpu as pltpu
from jax.experimental.pallas import tpu_sc as plsc

H = 1024
DFF = 512
E = 8
K = 2
SCALE = 2.5
T = 2048
B = 256
NBR = 24
PR = NBR * B


def _router_body(x_ref, gw_ref, w_ref, i_ref):
    x = x_ref[...]
    logits = lax.dot_general(x, gw_ref[...], (((1,), (1,)), ((), ())),
                             preferred_element_type=jnp.float32)
    v = jax.nn.sigmoid(logits)
    lane = lax.broadcasted_iota(jnp.int32, v.shape, 1)
    m1 = jnp.max(v, axis=1, keepdims=True)
    i1 = jnp.min(jnp.where(v == m1, lane, E), axis=1, keepdims=True)
    vm = jnp.where(lane == i1, -jnp.inf, v)
    m2 = jnp.max(vm, axis=1, keepdims=True)
    i2 = jnp.min(jnp.where(vm == m2, lane, E), axis=1, keepdims=True)
    s = m1 + m2 + 1e-6
    w_ref[...] = jnp.concatenate([m1 / s, m2 / s], axis=1) * SCALE
    i_ref[...] = jnp.concatenate([i1, i2], axis=1)


def _router(x, gate_w):
    return pl.pallas_call(
        _router_body,
        out_shape=(jax.ShapeDtypeStruct((T, K), jnp.float32),
                   jax.ShapeDtypeStruct((T, K), jnp.int32)),
    )(x, gate_w)


def _make_sc_gather(n_rows, n_chunks, name):
    info = plsc.get_sparse_core_info()
    nw = info.num_cores * info.num_subcores
    n_w = n_rows // nw
    chunk = n_w // n_chunks
    mesh = plsc.VectorSubcoreMesh(core_axis_name="c", subcore_axis_name="s")

    @functools.partial(
        pl.kernel, mesh=mesh, name=name,
        out_type=jax.ShapeDtypeStruct((n_rows, H), jnp.float32),
        scratch_types=[
            pltpu.VMEM((n_w,), jnp.int32),
            pltpu.VMEM((chunk, H), jnp.float32),
            pltpu.VMEM((chunk, H), jnp.float32),
            pltpu.SemaphoreType.DMA,
            pltpu.SemaphoreType.DMA,
        ],
    )
    def gather_kernel(src_hbm, idx_hbm, out_hbm, idx_v, rows0, rows1, s0, s1):
        wid = lax.axis_index("s") * info.num_cores + lax.axis_index("c")
        base = wid * n_w
        bufs, sems = (rows0, rows1), (s0, s1)
        pltpu.sync_copy(idx_hbm.at[pl.ds(base, n_w)], idx_v)

        def start(c):
            return pltpu.async_copy(
                src_hbm.at[idx_v.at[pl.ds(c * chunk, chunk)]],
                bufs[c % 2], sems[c % 2])

        cps = [None] * n_chunks
        cps[0] = start(0)
        if n_chunks > 1:
            cps[1] = start(1)
        for c in range(n_chunks):
            cps[c].wait()
            pltpu.sync_copy(bufs[c % 2],
                            out_hbm.at[pl.ds(base + c * chunk, chunk)])
            if c + 2 < n_chunks:
                cps[c + 2] = start(c + 2)

    return gather_kernel


def _mlp_body(x_ref, gw_ref, uw_ref, dw_ref, y_ref):
    xb = x_ref[...].astype(jnp.bfloat16)
    g = lax.dot_general(xb, gw_ref[0], (((1,), (1,)), ((), ())),
                        preferred_element_type=jnp.float32)
    u = lax.dot_general(xb, uw_ref[0], (((1,), (1,)), ((), ())),
                        preferred_element_type=jnp.float32)
    h = (jax.nn.silu(g) * u).astype(jnp.bfloat16)
    y_ref[...] = lax.dot_general(h, dw_ref[0], (((1,), (1,)), ((), ())),
                                 preferred_element_type=jnp.float32)


def _gmm(block_expert, xg, gw, uw, dw):
    grid_spec = pltpu.PrefetchScalarGridSpec(
        num_scalar_prefetch=1,
        grid=(NBR,),
        in_specs=[
            pl.BlockSpec((B, H), lambda i, be: (i, 0)),
            pl.BlockSpec((1, DFF, H), lambda i, be: (jnp.maximum(be[i], 0), 0, 0)),
            pl.BlockSpec((1, DFF, H), lambda i, be: (jnp.maximum(be[i], 0), 0, 0)),
            pl.BlockSpec((1, H, DFF), lambda i, be: (jnp.maximum(be[i], 0), 0, 0)),
        ],
        out_specs=pl.BlockSpec((B, H), lambda i, be: (i, 0)),
    )

    def body(be_ref, x, g, u, d, y):
        @pl.when(be_ref[pl.program_id(0)] >= 0)
        def _():
            _mlp_body(x, g, u, d, y)

    return pl.pallas_call(
        body,
        grid_spec=grid_spec,
        out_shape=jax.ShapeDtypeStruct((PR, H), jnp.float32),
    )(block_expert, xg, gw, uw, dw)


def _shared_mlp(x, sgw, suw, sdw):
    return pl.pallas_call(
        _mlp_body,
        grid=(T // B,),
        in_specs=[
            pl.BlockSpec((B, H), lambda i: (i, 0)),
            pl.BlockSpec((1, DFF, H), lambda i: (0, 0, 0)),
            pl.BlockSpec((1, DFF, H), lambda i: (0, 0, 0)),
            pl.BlockSpec((1, H, DFF), lambda i: (0, 0, 0)),
        ],
        out_specs=pl.BlockSpec((B, H), lambda i: (i, 0)),
        out_shape=jax.ShapeDtypeStruct((T, H), jnp.float32),
    )(x, sgw, suw, sdw)


def _combine_body(g0_ref, g1_ref, sh_ref, w_ref, o_ref):
    w = w_ref[...]
    o_ref[...] = (w[:, 0:1] * g0_ref[...]
                  + w[:, 1:2] * g1_ref[...]
                  + sh_ref[...])


def _combine(g, sh, topk_w):
    bt = 256
    return pl.pallas_call(
        _combine_body,
        grid=(T // bt,),
        in_specs=[pl.BlockSpec((bt, H), lambda i: (i, 0)),
                  pl.BlockSpec((bt, H), lambda i: (i + T // bt, 0)),
                  pl.BlockSpec((bt, H), lambda i: (i, 0)),
                  pl.BlockSpec((bt, K), lambda i: (i, 0))],
        out_specs=pl.BlockSpec((bt, H), lambda i: (i, 0)),
        out_shape=jax.ShapeDtypeStruct((T, H), jnp.float32),
    )(g, g, sh, topk_w)


def kernel(hidden_states, gate_w, shared_gate_w, shared_up_w, shared_down_w,
           expert_gate_w, expert_up_w, expert_down_w):
    orig_shape = hidden_states.shape
    x = hidden_states.reshape(-1, H)

    topk_w, topk_i = _router(x, gate_w)

    flat_e = topk_i.reshape(-1)
    onehot = (flat_e[:, None] == jnp.arange(E)[None, :]).astype(jnp.int32)
    incl = jnp.cumsum(onehot, axis=0)
    counts = incl[-1]
    pos = incl[jnp.arange(T * K), flat_e] - 1
    pad_counts = ((counts + B - 1) // B) * B
    pad_off = jnp.concatenate([jnp.zeros((1,), jnp.int32),
                               jnp.cumsum(pad_counts)]).astype(jnp.int32)
    dest = pad_off[flat_e] + pos

    sorted_token = (jnp.arange(PR, dtype=jnp.int32) % T).at[dest].set(
        jnp.arange(T * K, dtype=jnp.int32) // K)

    b_start = jnp.arange(NBR, dtype=jnp.int32) * B
    block_expert = jnp.where(
        b_start < pad_off[E],
        jnp.minimum(
            jnp.sum((b_start[:, None] >= pad_off[None, 1:E + 1])
                    .astype(jnp.int32), axis=1), E - 1),
        -1)

    xg = _make_sc_gather(PR, 4, "sc_gather_dispatch")(x, sorted_token)

    yg = _gmm(block_expert, xg,
              expert_gate_w.astype(jnp.bfloat16),
              expert_up_w.astype(jnp.bfloat16),
              expert_down_w.astype(jnp.bfloat16))

    sh = _shared_mlp(x, shared_gate_w[None].astype(jnp.bfloat16),
                     shared_up_w[None].astype(jnp.bfloat16),
                     shared_down_w[None].astype(jnp.bfloat16))

    d_planar = dest.reshape(T, K).T.reshape(-1)
    g = _make_sc_gather(T * K, 4, "sc_gather_combine")(yg, d_planar)

    out = _combine(g, sh, topk_w)
    return out.reshape(orig_shape)


# speedup vs baseline: 2.1819x; 1.1481x over previous
import functools

import jax
import jax.numpy as jnp
from jax import lax
from jax.experimental import pallas as pl
from jax.experimental.pallas import tpu as pltpu
from jax.experimental.pallas import tpu_sc as plsc

H = 1024
DFF = 512
E = 8
K = 2
SCALE = 2.5
T = 2048
B = 256
NBR = 24
PR = NBR * B


def _router_body(x_ref, gw_ref, w_ref, i_ref):
    x = x_ref[...]
    logits = lax.dot_general(x, gw_ref[...], (((1,), (1,)), ((), ())),
                             preferred_element_type=jnp.float32)
    v = jax.nn.sigmoid(logits)
    lane = lax.broadcasted_iota(jnp.int32, v.shape, 1)
    m1 = jnp.max(v, axis=1, keepdims=True)
    i1 = jnp.min(jnp.where(v == m1, lane, E), axis=1, keepdims=True)
    vm = jnp.where(lane == i1, -jnp.inf, v)
    m2 = jnp.max(vm, axis=1, keepdims=True)
    i2 = jnp.min(jnp.where(vm == m2, lane, E), axis=1, keepdims=True)
    s = m1 + m2 + 1e-6
    w_ref[...] = jnp.concatenate([m1 / s, m2 / s], axis=1) * SCALE
    i_ref[...] = jnp.concatenate([i1, i2], axis=1)


def _router(x, gate_w):
    return pl.pallas_call(
        _router_body,
        out_shape=(jax.ShapeDtypeStruct((T, K), jnp.float32),
                   jax.ShapeDtypeStruct((T, K), jnp.int32)),
    )(x, gate_w)


def _make_sc_dispatch():
    info = plsc.get_sparse_core_info()
    nw = info.num_cores * info.num_subcores
    nt = T // nw
    mesh = plsc.VectorSubcoreMesh(core_axis_name="c", subcore_axis_name="s")

    @functools.partial(
        pl.kernel, mesh=mesh, name="sc_dispatch_scatter",
        out_type=jax.ShapeDtypeStruct((PR, H), jnp.float32),
        scratch_types=[
            pltpu.VMEM((nt, H), jnp.float32),
            pltpu.VMEM((nt,), jnp.int32),
            pltpu.VMEM((nt,), jnp.int32),
            pltpu.SemaphoreType.DMA,
        ],
    )
    def dispatch_kernel(x_hbm, d0_hbm, d1_hbm, out_hbm, xv, i0v, i1v, sem):
        wid = lax.axis_index("s") * info.num_cores + lax.axis_index("c")
        base = wid * nt
        pltpu.sync_copy(x_hbm.at[pl.ds(base, nt)], xv)
        pltpu.sync_copy(d0_hbm.at[pl.ds(base, nt)], i0v)
        pltpu.sync_copy(d1_hbm.at[pl.ds(base, nt)], i1v)
        c0 = pltpu.async_copy(xv, out_hbm.at[i0v], sem)
        c1 = pltpu.async_copy(xv, out_hbm.at[i1v], sem)
        c0.wait()
        c1.wait()

    return dispatch_kernel


def _make_sc_gather(n_rows, n_chunks, name):
    info = plsc.get_sparse_core_info()
    nw = info.num_cores * info.num_subcores
    n_w = n_rows // nw
    chunk = n_w // n_chunks
    mesh = plsc.VectorSubcoreMesh(core_axis_name="c", subcore_axis_name="s")

    @functools.partial(
        pl.kernel, mesh=mesh, name=name,
        out_type=jax.ShapeDtypeStruct((n_rows, H), jnp.float32),
        scratch_types=[
            pltpu.VMEM((n_w,), jnp.int32),
            pltpu.VMEM((chunk, H), jnp.float32),
            pltpu.VMEM((chunk, H), jnp.float32),
            pltpu.SemaphoreType.DMA,
            pltpu.SemaphoreType.DMA,
        ],
    )
    def gather_kernel(src_hbm, idx_hbm, out_hbm, idx_v, rows0, rows1, s0, s1):
        wid = lax.axis_index("s") * info.num_cores + lax.axis_index("c")
        base = wid * n_w
        bufs, sems = (rows0, rows1), (s0, s1)
        pltpu.sync_copy(idx_hbm.at[pl.ds(base, n_w)], idx_v)

        def start(c):
            return pltpu.async_copy(
                src_hbm.at[idx_v.at[pl.ds(c * chunk, chunk)]],
                bufs[c % 2], sems[c % 2])

        cps = [None] * n_chunks
        cps[0] = start(0)
        if n_chunks > 1:
            cps[1] = start(1)
        for c in range(n_chunks):
            cps[c].wait()
            pltpu.sync_copy(bufs[c % 2],
                            out_hbm.at[pl.ds(base + c * chunk, chunk)])
            if c + 2 < n_chunks:
                cps[c + 2] = start(c + 2)

    return gather_kernel


def _mlp_body(x_ref, gw_ref, uw_ref, dw_ref, y_ref):
    xb = x_ref[...].astype(jnp.bfloat16)
    g = lax.dot_general(xb, gw_ref[0], (((1,), (1,)), ((), ())),
                        preferred_element_type=jnp.float32)
    u = lax.dot_general(xb, uw_ref[0], (((1,), (1,)), ((), ())),
                        preferred_element_type=jnp.float32)
    h = (jax.nn.silu(g) * u).astype(jnp.bfloat16)
    y_ref[...] = lax.dot_general(h, dw_ref[0], (((1,), (1,)), ((), ())),
                                 preferred_element_type=jnp.float32)


def _gmm(block_expert, xg, gw, uw, dw):
    grid_spec = pltpu.PrefetchScalarGridSpec(
        num_scalar_prefetch=1,
        grid=(NBR,),
        in_specs=[
            pl.BlockSpec((B, H), lambda i, be: (i, 0)),
            pl.BlockSpec((1, DFF, H), lambda i, be: (jnp.maximum(be[i], 0), 0, 0)),
            pl.BlockSpec((1, DFF, H), lambda i, be: (jnp.maximum(be[i], 0), 0, 0)),
            pl.BlockSpec((1, H, DFF), lambda i, be: (jnp.maximum(be[i], 0), 0, 0)),
        ],
        out_specs=pl.BlockSpec((B, H), lambda i, be: (i, 0)),
    )

    def body(be_ref, x, g, u, d, y):
        @pl.when(be_ref[pl.program_id(0)] >= 0)
        def _():
            _mlp_body(x, g, u, d, y)

    return pl.pallas_call(
        body,
        grid_spec=grid_spec,
        out_shape=jax.ShapeDtypeStruct((PR, H), jnp.float32),
    )(block_expert, xg, gw, uw, dw)


def _shared_mlp(x, sgw, suw, sdw):
    return pl.pallas_call(
        _mlp_body,
        grid=(T // B,),
        in_specs=[
            pl.BlockSpec((B, H), lambda i: (i, 0)),
            pl.BlockSpec((1, DFF, H), lambda i: (0, 0, 0)),
            pl.BlockSpec((1, DFF, H), lambda i: (0, 0, 0)),
            pl.BlockSpec((1, H, DFF), lambda i: (0, 0, 0)),
        ],
        out_specs=pl.BlockSpec((B, H), lambda i: (i, 0)),
        out_shape=jax.ShapeDtypeStruct((T, H), jnp.float32),
    )(x, sgw, suw, sdw)


def _combine_body(g0_ref, g1_ref, sh_ref, w_ref, o_ref):
    w = w_ref[...]
    o_ref[...] = (w[:, 0:1] * g0_ref[...]
                  + w[:, 1:2] * g1_ref[...]
                  + sh_ref[...])


def _combine(g, sh, topk_w):
    bt = 256
    return pl.pallas_call(
        _combine_body,
        grid=(T // bt,),
        in_specs=[pl.BlockSpec((bt, H), lambda i: (i, 0)),
                  pl.BlockSpec((bt, H), lambda i: (i + T // bt, 0)),
                  pl.BlockSpec((bt, H), lambda i: (i, 0)),
                  pl.BlockSpec((bt, K), lambda i: (i, 0))],
        out_specs=pl.BlockSpec((bt, H), lambda i: (i, 0)),
        out_shape=jax.ShapeDtypeStruct((T, H), jnp.float32),
    )(g, g, sh, topk_w)


def kernel(hidden_states, gate_w, shared_gate_w, shared_up_w, shared_down_w,
           expert_gate_w, expert_up_w, expert_down_w):
    orig_shape = hidden_states.shape
    x = hidden_states.reshape(-1, H)

    topk_w, topk_i = _router(x, gate_w)

    flat_e = topk_i.reshape(-1)
    onehot = (flat_e[:, None] == jnp.arange(E)[None, :]).astype(jnp.int32)
    incl = jnp.cumsum(onehot, axis=0)
    counts = incl[-1]
    pos = incl[jnp.arange(T * K), flat_e] - 1
    pad_counts = ((counts + B - 1) // B) * B
    pad_off = jnp.concatenate([jnp.zeros((1,), jnp.int32),
                               jnp.cumsum(pad_counts)]).astype(jnp.int32)
    dest = pad_off[flat_e] + pos

    b_start = jnp.arange(NBR, dtype=jnp.int32) * B
    block_expert = jnp.where(
        b_start < pad_off[E],
        jnp.minimum(
            jnp.sum((b_start[:, None] >= pad_off[None, 1:E + 1])
                    .astype(jnp.int32), axis=1), E - 1),
        -1)

    d_cols = dest.reshape(T, K)
    xg = _make_sc_dispatch()(x, d_cols[:, 0], d_cols[:, 1])

    yg = _gmm(block_expert, xg,
              expert_gate_w.astype(jnp.bfloat16),
              expert_up_w.astype(jnp.bfloat16),
              expert_down_w.astype(jnp.bfloat16))

    sh = _shared_mlp(x, shared_gate_w[None].astype(jnp.bfloat16),
                     shared_up_w[None].astype(jnp.bfloat16),
                     shared_down_w[None].astype(jnp.bfloat16))

    d_planar = dest.reshape(T, K).T.reshape(-1)
    g = _make_sc_gather(T * K, 4, "sc_gather_combine")(yg, d_planar)

    out = _combine(g, sh, topk_w)
    return out.reshape(orig_shape)


# speedup vs baseline: 2.2690x; 1.0399x over previous
import functools

import jax
import jax.numpy as jnp
from jax import lax
from jax.experimental import pallas as pl
from jax.experimental.pallas import tpu as pltpu
from jax.experimental.pallas import tpu_sc as plsc

H = 1024
DFF = 512
E = 8
K = 2
SCALE = 2.5
T = 2048
B = 256
NBR = 24
PR = NBR * B


def _router_body(x_ref, gw_ref, w_ref, i_ref):
    x = x_ref[...]
    logits = lax.dot_general(x, gw_ref[...], (((1,), (1,)), ((), ())),
                             preferred_element_type=jnp.float32)
    v = jax.nn.sigmoid(logits)
    lane = lax.broadcasted_iota(jnp.int32, v.shape, 1)
    m1 = jnp.max(v, axis=1, keepdims=True)
    i1 = jnp.min(jnp.where(v == m1, lane, E), axis=1, keepdims=True)
    vm = jnp.where(lane == i1, -jnp.inf, v)
    m2 = jnp.max(vm, axis=1, keepdims=True)
    i2 = jnp.min(jnp.where(vm == m2, lane, E), axis=1, keepdims=True)
    s = m1 + m2 + 1e-6
    w_ref[...] = jnp.concatenate([m1 / s, m2 / s], axis=1) * SCALE
    i_ref[...] = jnp.concatenate([i1, i2], axis=1)


def _router(x, gate_w):
    return pl.pallas_call(
        _router_body,
        out_shape=(jax.ShapeDtypeStruct((T, K), jnp.float32),
                   jax.ShapeDtypeStruct((T, K), jnp.int32)),
    )(x, gate_w)


def _make_sc_dispatch():
    info = plsc.get_sparse_core_info()
    nw = info.num_cores * info.num_subcores
    nt = T // nw
    mesh = plsc.VectorSubcoreMesh(core_axis_name="c", subcore_axis_name="s")

    @functools.partial(
        pl.kernel, mesh=mesh, name="sc_dispatch_scatter",
        out_type=jax.ShapeDtypeStruct((PR, H), jnp.float32),
        scratch_types=[
            pltpu.VMEM((nt, H), jnp.float32),
            pltpu.VMEM((nt,), jnp.int32),
            pltpu.VMEM((nt,), jnp.int32),
            pltpu.SemaphoreType.DMA,
        ],
    )
    def dispatch_kernel(x_hbm, d0_hbm, d1_hbm, out_hbm, xv, i0v, i1v, sem):
        wid = lax.axis_index("s") * info.num_cores + lax.axis_index("c")
        base = wid * nt
        pltpu.sync_copy(x_hbm.at[pl.ds(base, nt)], xv)
        pltpu.sync_copy(d0_hbm.at[pl.ds(base, nt)], i0v)
        pltpu.sync_copy(d1_hbm.at[pl.ds(base, nt)], i1v)
        c0 = pltpu.async_copy(xv, out_hbm.at[i0v], sem)
        c1 = pltpu.async_copy(xv, out_hbm.at[i1v], sem)
        c0.wait()
        c1.wait()

    return dispatch_kernel


def _make_sc_gather(n_rows, n_chunks, name, dtype=jnp.float32):
    info = plsc.get_sparse_core_info()
    nw = info.num_cores * info.num_subcores
    n_w = n_rows // nw
    chunk = n_w // n_chunks
    mesh = plsc.VectorSubcoreMesh(core_axis_name="c", subcore_axis_name="s")

    @functools.partial(
        pl.kernel, mesh=mesh, name=name,
        out_type=jax.ShapeDtypeStruct((n_rows, H), dtype),
        scratch_types=[
            pltpu.VMEM((n_w,), jnp.int32),
            pltpu.VMEM((chunk, H), dtype),
            pltpu.VMEM((chunk, H), dtype),
            pltpu.SemaphoreType.DMA,
            pltpu.SemaphoreType.DMA,
        ],
    )
    def gather_kernel(src_hbm, idx_hbm, out_hbm, idx_v, rows0, rows1, s0, s1):
        wid = lax.axis_index("s") * info.num_cores + lax.axis_index("c")
        base = wid * n_w
        bufs, sems = (rows0, rows1), (s0, s1)
        pltpu.sync_copy(idx_hbm.at[pl.ds(base, n_w)], idx_v)

        def start(c):
            return pltpu.async_copy(
                src_hbm.at[idx_v.at[pl.ds(c * chunk, chunk)]],
                bufs[c % 2], sems[c % 2])

        cps = [None] * n_chunks
        cps[0] = start(0)
        if n_chunks > 1:
            cps[1] = start(1)
        for c in range(n_chunks):
            cps[c].wait()
            pltpu.sync_copy(bufs[c % 2],
                            out_hbm.at[pl.ds(base + c * chunk, chunk)])
            if c + 2 < n_chunks:
                cps[c + 2] = start(c + 2)

    return gather_kernel


NBS = NBR + T // B


def _mlp_compute(xb, gw16, uw16, dw16, y_ref):
    g = lax.dot_general(xb, gw16[...], (((1,), (1,)), ((), ())),
                        preferred_element_type=jnp.float32)
    u = lax.dot_general(xb, uw16[...], (((1,), (1,)), ((), ())),
                        preferred_element_type=jnp.float32)
    h = (jax.nn.silu(g) * u).astype(jnp.bfloat16)
    y_ref[...] = lax.dot_general(h, dw16[...], (((1,), (1,)), ((), ())),
                                 preferred_element_type=jnp.float32)


def _gmm(block_expert, xg, x, gw, uw, dw, sgw, suw, sdw):
    grid_spec = pltpu.PrefetchScalarGridSpec(
        num_scalar_prefetch=1,
        grid=(NBS,),
        in_specs=[
            pl.BlockSpec((B, H), lambda i, be: (jnp.minimum(i, NBR - 1), 0)),
            pl.BlockSpec((B, H), lambda i, be: (jnp.maximum(i - NBR, 0), 0)),
            pl.BlockSpec((1, DFF, H),
                         lambda i, be: (jnp.clip(be[i], 0, E - 1), 0, 0)),
            pl.BlockSpec((1, DFF, H),
                         lambda i, be: (jnp.clip(be[i], 0, E - 1), 0, 0)),
            pl.BlockSpec((1, H, DFF),
                         lambda i, be: (jnp.clip(be[i], 0, E - 1), 0, 0)),
            pl.BlockSpec((1, DFF, H), lambda i, be: (0, 0, 0)),
            pl.BlockSpec((1, DFF, H), lambda i, be: (0, 0, 0)),
            pl.BlockSpec((1, H, DFF), lambda i, be: (0, 0, 0)),
        ],
        out_specs=pl.BlockSpec((B, H), lambda i, be: (i, 0)),
        scratch_shapes=[
            pltpu.VMEM((DFF, H), jnp.bfloat16),
            pltpu.VMEM((DFF, H), jnp.bfloat16),
            pltpu.VMEM((H, DFF), jnp.bfloat16),
        ],
    )

    def body(be_ref, xg_ref, x_ref, gw_ref, uw_ref, dw_ref,
             sgw_ref, suw_ref, sdw_ref, y_ref, gw16, uw16, dw16):
        i = pl.program_id(0)
        e = be_ref[i]
        prev = jnp.where(i == 0, -2, be_ref[jnp.maximum(i - 1, 0)])

        @pl.when((e != prev) & (e >= 0))
        def _cast():
            @pl.when(e < E)
            def _():
                gw16[...] = gw_ref[0].astype(jnp.bfloat16)
                uw16[...] = uw_ref[0].astype(jnp.bfloat16)
                dw16[...] = dw_ref[0].astype(jnp.bfloat16)

            @pl.when(e == E)
            def _():
                gw16[...] = sgw_ref[0].astype(jnp.bfloat16)
                uw16[...] = suw_ref[0].astype(jnp.bfloat16)
                dw16[...] = sdw_ref[0].astype(jnp.bfloat16)

        @pl.when((e >= 0) & (e < E))
        def _routed():
            _mlp_compute(xg_ref[...].astype(jnp.bfloat16),
                         gw16, uw16, dw16, y_ref)

        @pl.when(e == E)
        def _shared():
            _mlp_compute(x_ref[...].astype(jnp.bfloat16),
                         gw16, uw16, dw16, y_ref)

    return pl.pallas_call(
        body,
        grid_spec=grid_spec,
        out_shape=jax.ShapeDtypeStruct((PR + T, H), jnp.float32),
    )(block_expert, xg, x, gw, uw, dw, sgw, suw, sdw)


def _combine_body(g0_ref, g1_ref, sh_ref, w_ref, o_ref):
    w = w_ref[...]
    o_ref[...] = (w[:, 0:1] * g0_ref[...].astype(jnp.float32)
                  + w[:, 1:2] * g1_ref[...].astype(jnp.float32)
                  + sh_ref[...])


def _combine(g, yg, topk_w):
    bt = 256
    return pl.pallas_call(
        _combine_body,
        grid=(T // bt,),
        in_specs=[pl.BlockSpec((bt, H), lambda i: (i, 0)),
                  pl.BlockSpec((bt, H), lambda i: (i + T // bt, 0)),
                  pl.BlockSpec((bt, H), lambda i: (i + PR // bt, 0)),
                  pl.BlockSpec((bt, K), lambda i: (i, 0))],
        out_specs=pl.BlockSpec((bt, H), lambda i: (i, 0)),
        out_shape=jax.ShapeDtypeStruct((T, H), jnp.float32),
    )(g, g, yg, topk_w)


def kernel(hidden_states, gate_w, shared_gate_w, shared_up_w, shared_down_w,
           expert_gate_w, expert_up_w, expert_down_w):
    orig_shape = hidden_states.shape
    x = hidden_states.reshape(-1, H)

    topk_w, topk_i = _router(x, gate_w)

    flat_e = topk_i.reshape(-1)
    onehot = (flat_e[:, None] == jnp.arange(E)[None, :]).astype(jnp.int32)
    incl = jnp.cumsum(onehot, axis=0)
    counts = incl[-1]
    pos = incl[jnp.arange(T * K), flat_e] - 1
    pad_counts = ((counts + B - 1) // B) * B
    pad_off = jnp.concatenate([jnp.zeros((1,), jnp.int32),
                               jnp.cumsum(pad_counts)]).astype(jnp.int32)
    dest = pad_off[flat_e] + pos

    b_start = jnp.arange(NBR, dtype=jnp.int32) * B
    block_expert = jnp.where(
        b_start < pad_off[E],
        jnp.minimum(
            jnp.sum((b_start[:, None] >= pad_off[None, 1:E + 1])
                    .astype(jnp.int32), axis=1), E - 1),
        -1)
    block_expert = jnp.concatenate(
        [block_expert, jnp.full((T // B,), E, jnp.int32)])

    d_cols = dest.reshape(T, K)
    xg = _make_sc_dispatch()(x, d_cols[:, 0], d_cols[:, 1])

    yg = _gmm(block_expert, xg, x,
              expert_gate_w, expert_up_w, expert_down_w,
              shared_gate_w[None], shared_up_w[None], shared_down_w[None])

    d_planar = dest.reshape(T, K).T.reshape(-1)
    g = _make_sc_gather(T * K, 4, "sc_gather_combine")(yg, d_planar)

    out = _combine(g, yg, topk_w)
    return out.reshape(orig_shape)
